# Initial kernel scaffold; baseline (speedup 1.0000x reference)
import jax
import jax.numpy as jnp
from jax.experimental import pallas as pl


def kernel(x, edge_index, batch, W_gat, a_src, a_dst, b_gat, W_gcn, b_gcn, W_fc1, b_fc1, W_fc2, b_fc2):
    raise NotImplementedError("write your pallas kernel here")


# speedup vs baseline: 1.9533x; 1.9533x over previous
import functools

import jax
import jax.numpy as jnp
from jax.experimental import pallas as pl
from jax.experimental.pallas import tpu as pltpu

H = 10
F_IN = 78


def _mm_kernel(x_ref, w_ref, b_ref, o_ref, acc_ref, *, nk, act):
    @pl.when(pl.program_id(2) == 0)
    def _init():
        acc_ref[...] = jnp.zeros_like(acc_ref)

    acc_ref[...] += jnp.dot(
        x_ref[...], w_ref[...], preferred_element_type=jnp.float32
    )

    @pl.when(pl.program_id(2) == nk - 1)
    def _done():
        r = acc_ref[...] + b_ref[...]
        if act == "relu":
            r = jnp.maximum(r, 0.0)
        o_ref[...] = r


def _mm(x, w, b=None, act=None, bm=256, bn=256, bk=128):
    M, K = x.shape
    _, N = w.shape
    bm = min(bm, -(-M // 8) * 8)
    Mp = -(-M // bm) * bm
    Kp = -(-K // bk) * bk
    Np = -(-N // bn) * bn
    xp = jnp.pad(x, ((0, Mp - M), (0, Kp - K)))
    wp = jnp.pad(w, ((0, Kp - K), (0, Np - N)))
    if b is None:
        bp = jnp.zeros((1, Np), jnp.float32)
    else:
        bp = jnp.pad(b.reshape(1, -1), ((0, 0), (0, Np - N)))
    nk = Kp // bk
    out = pl.pallas_call(
        functools.partial(_mm_kernel, nk=nk, act=act),
        grid=(Mp // bm, Np // bn, nk),
        in_specs=[
            pl.BlockSpec((bm, bk), lambda i, j, k: (i, k)),
            pl.BlockSpec((bk, bn), lambda i, j, k: (k, j)),
            pl.BlockSpec((1, bn), lambda i, j, k: (0, j)),
        ],
        out_specs=pl.BlockSpec((bm, bn), lambda i, j, k: (i, j)),
        out_shape=jax.ShapeDtypeStruct((Mp, Np), jnp.float32),
        scratch_shapes=[pltpu.VMEM((bm, bn), jnp.float32)],
    )(xp, wp, bp)
    return out[:M, :N]


def kernel(x, edge_index, batch, W_gat, a_src, a_dst, b_gat, W_gcn, b_gcn,
           W_fc1, b_fc1, W_fc2, b_fc2):
    N = x.shape[0]
    B = 256
    dh = H * F_IN
    loop = jnp.arange(N, dtype=jnp.int32)
    src = jnp.concatenate([edge_index[0].astype(jnp.int32), loop])
    dst = jnp.concatenate([edge_index[1].astype(jnp.int32), loop])
    batch = batch.astype(jnp.int32)

    h = _mm(x, W_gat)
    rows = jnp.arange(dh, dtype=jnp.int32)
    A = jnp.zeros((dh, 2 * H), jnp.float32)
    A = A.at[rows, rows // F_IN].set(a_src.reshape(-1))
    A = A.at[rows, H + rows // F_IN].set(a_dst.reshape(-1))
    al = _mm(h, A)
    al_s, al_d = al[:, :H], al[:, H:]

    e = al_s[src] + al_d[dst]
    e = jnp.where(e >= 0, e, 0.2 * e)
    m = jax.ops.segment_max(e, dst, num_segments=N)
    m = jnp.where(jnp.isfinite(m), m, 0.0)
    ee = jnp.exp(e - m[dst])
    denom = jax.ops.segment_sum(ee, dst, num_segments=N)
    alpha = ee / (denom[dst] + 1e-16)
    w_edge = jnp.repeat(alpha, F_IN, axis=1)
    drug = jax.ops.segment_sum(h[src] * w_edge, dst, num_segments=N) + b_gat
    drug = jnp.maximum(drug, 0.0)

    deg = jax.ops.segment_sum(jnp.ones_like(src, dtype=jnp.float32), dst,
                              num_segments=N)
    dinv = jnp.where(deg > 0, jax.lax.rsqrt(deg), 0.0)
    norm = dinv[src] * dinv[dst]
    h2 = _mm(drug, W_gcn)
    drug = jax.ops.segment_sum(h2[src] * norm[:, None], dst,
                               num_segments=N) + b_gcn
    drug = jnp.maximum(drug, 0.0)

    counts = jax.ops.segment_sum(jnp.ones((N,), jnp.float32), batch,
                                 num_segments=B)
    mean_p = jax.ops.segment_sum(drug, batch, num_segments=B)
    mean_p = mean_p / jnp.clip(counts, 1.0)[:, None]
    max_p = jax.ops.segment_max(drug, batch, num_segments=B)
    max_p = jnp.where(counts[:, None] > 0, max_p, 0.0)
    g = jnp.concatenate([mean_p, max_p], axis=1)

    g = _mm(g, W_fc1, b_fc1, act="relu")
    g = _mm(g, W_fc2, b_fc2)
    return g


# speedup vs baseline: 1.9870x; 1.0173x over previous
import functools

import jax
import jax.numpy as jnp
from jax.experimental import pallas as pl
from jax.experimental.pallas import tpu as pltpu

H = 10
F_IN = 78


def _mm_kernel(x_ref, w_ref, b_ref, o_ref, acc_ref, *, nk, act):
    @pl.when(pl.program_id(2) == 0)
    def _init():
        acc_ref[...] = jnp.zeros_like(acc_ref)

    acc_ref[...] += jnp.dot(
        x_ref[...], w_ref[...], preferred_element_type=jnp.float32
    )

    @pl.when(pl.program_id(2) == nk - 1)
    def _done():
        r = acc_ref[...] + b_ref[...]
        if act == "relu":
            r = jnp.maximum(r, 0.0)
        o_ref[...] = r


def _mm(x, w, b=None, act=None, bm=256, bn=256, bk=128):
    M, K = x.shape
    _, N = w.shape
    bm = min(bm, -(-M // 8) * 8)
    Mp = -(-M // bm) * bm
    Kp = -(-K // bk) * bk
    Np = -(-N // bn) * bn
    xp = jnp.pad(x, ((0, Mp - M), (0, Kp - K)))
    wp = jnp.pad(w, ((0, Kp - K), (0, Np - N)))
    if b is None:
        bp = jnp.zeros((1, Np), jnp.float32)
    else:
        bp = jnp.pad(b.reshape(1, -1), ((0, 0), (0, Np - N)))
    nk = Kp // bk
    out = pl.pallas_call(
        functools.partial(_mm_kernel, nk=nk, act=act),
        grid=(Mp // bm, Np // bn, nk),
        in_specs=[
            pl.BlockSpec((bm, bk), lambda i, j, k: (i, k)),
            pl.BlockSpec((bk, bn), lambda i, j, k: (k, j)),
            pl.BlockSpec((1, bn), lambda i, j, k: (0, j)),
        ],
        out_specs=pl.BlockSpec((bm, bn), lambda i, j, k: (i, j)),
        out_shape=jax.ShapeDtypeStruct((Mp, Np), jnp.float32),
        scratch_shapes=[pltpu.VMEM((bm, bn), jnp.float32)],
    )(xp, wp, bp)
    return out[:M, :N]


def kernel(x, edge_index, batch, W_gat, a_src, a_dst, b_gat, W_gcn, b_gcn,
           W_fc1, b_fc1, W_fc2, b_fc2):
    N = x.shape[0]
    B = 256
    dh = H * F_IN
    loop = jnp.arange(N, dtype=jnp.int32)
    src = jnp.concatenate([edge_index[0].astype(jnp.int32), loop])
    dst = jnp.concatenate([edge_index[1].astype(jnp.int32), loop])
    order = jnp.argsort(dst)
    src = src[order]
    dst = dst[order]
    batch = batch.astype(jnp.int32)

    h = _mm(x, W_gat)
    rows = jnp.arange(dh, dtype=jnp.int32)
    A = jnp.zeros((dh, 2 * H), jnp.float32)
    A = A.at[rows, rows // F_IN].set(a_src.reshape(-1))
    A = A.at[rows, H + rows // F_IN].set(a_dst.reshape(-1))
    al = _mm(h, A)
    al_s, al_d = al[:, :H], al[:, H:]

    e = al_s[src] + al_d[dst]
    e = jnp.where(e >= 0, e, 0.2 * e)
    m = jax.ops.segment_max(e, dst, num_segments=N, indices_are_sorted=True)
    m = jnp.where(jnp.isfinite(m), m, 0.0)
    ee = jnp.exp(e - m[dst])
    denom = jax.ops.segment_sum(ee, dst, num_segments=N, indices_are_sorted=True)
    alpha = ee / (denom[dst] + 1e-16)
    w_edge = jnp.repeat(alpha, F_IN, axis=1)
    drug = jax.ops.segment_sum(h[src] * w_edge, dst, num_segments=N,
                               indices_are_sorted=True) + b_gat
    drug = jnp.maximum(drug, 0.0)

    deg = jax.ops.segment_sum(jnp.ones_like(src, dtype=jnp.float32), dst,
                              num_segments=N, indices_are_sorted=True)
    dinv = jnp.where(deg > 0, jax.lax.rsqrt(deg), 0.0)
    norm = dinv[src] * dinv[dst]
    h2 = _mm(drug, W_gcn)
    drug = jax.ops.segment_sum(h2[src] * norm[:, None], dst,
                               num_segments=N, indices_are_sorted=True) + b_gcn
    drug = jnp.maximum(drug, 0.0)

    counts = jax.ops.segment_sum(jnp.ones((N,), jnp.float32), batch,
                                 num_segments=B, indices_are_sorted=True)
    mean_p = jax.ops.segment_sum(drug, batch, num_segments=B,
                                 indices_are_sorted=True)
    mean_p = mean_p / jnp.clip(counts, 1.0)[:, None]
    max_p = jax.ops.segment_max(drug, batch, num_segments=B,
                                indices_are_sorted=True)
    max_p = jnp.where(counts[:, None] > 0, max_p, 0.0)
    g = jnp.concatenate([mean_p, max_p], axis=1)

    g = _mm(g, W_fc1, b_fc1, act="relu")
    g = _mm(g, W_fc2, b_fc2)
    return g
